# baseline (speedup 1.0000x reference)
import functools

import jax
import jax.numpy as jnp
from jax.experimental import pallas as pl
from jax.experimental.pallas import tpu as pltpu


def _hwnc_body(x_ref, o_ref, *, inv_hw):
    o_ref[...] = jnp.sum(x_ref[...], axis=0, dtype=jnp.float32) * inv_hw


def _gap_hwnc(x, N, C, hw):
    inv_hw = 1.0 / float(hw)
    xp = jnp.transpose(x, (2, 3, 0, 1)).reshape(hw, N, C)

    bn = 16 if N % 16 == 0 else (8 if N % 8 == 0 else N)
    bc = 1024 if C % 1024 == 0 else C
    out = pl.pallas_call(
        functools.partial(_hwnc_body, inv_hw=inv_hw),
        out_shape=jax.ShapeDtypeStruct((N, C), jnp.float32),
        grid=(N // bn, C // bc),
        in_specs=[pl.BlockSpec((hw, bn, bc), lambda i, j: (0, i, j))],
        out_specs=pl.BlockSpec((bn, bc), lambda i, j: (i, j)),
        compiler_params=pltpu.CompilerParams(
            dimension_semantics=("parallel", "parallel"),
            vmem_limit_bytes=64 << 20,
        ),
        cost_estimate=pl.CostEstimate(
            flops=N * C * hw, transcendentals=0,
            bytes_accessed=N * C * hw * 4 + N * C * 4),
    )(xp)
    return out.reshape(N, C, 1, 1)


def _nhwc_body(x_ref, o_ref, *, inv_hw):
    o_ref[...] = jnp.sum(x_ref[...], axis=1, keepdims=True,
                         dtype=jnp.float32) * inv_hw


def _gap_fallback(x, N, C, hw):
    inv_hw = 1.0 / float(hw)
    xt = jnp.transpose(x.reshape(N, C, hw), (0, 2, 1))
    out = pl.pallas_call(
        functools.partial(_nhwc_body, inv_hw=inv_hw),
        out_shape=jax.ShapeDtypeStruct((N, 1, C), jnp.float32),
        grid=(N,),
        in_specs=[pl.BlockSpec((1, hw, C), lambda i: (i, 0, 0))],
        out_specs=pl.BlockSpec((1, 1, C), lambda i: (i, 0, 0)),
        compiler_params=pltpu.CompilerParams(
            dimension_semantics=("parallel",),
            vmem_limit_bytes=64 << 20,
        ),
    )(xt)
    return out.reshape(N, C, 1, 1)


def kernel(x):
    N, C, H, W = x.shape
    hw = H * W
    if C % 128 == 0 and N % 8 == 0:
        return _gap_hwnc(x, N, C, hw)
    return _gap_fallback(x, N, C, hw)


# speedup vs baseline: 1.1244x; 1.1244x over previous
import functools

import jax
import jax.numpy as jnp
from jax.experimental import pallas as pl
from jax.experimental.pallas import tpu as pltpu


def _hwnc_body(x_ref, o_ref, *, inv_hw):
    o_ref[...] = jnp.sum(x_ref[...], axis=0, dtype=jnp.float32) * inv_hw


def _gap_hwnc(x, N, C, hw):
    inv_hw = 1.0 / float(hw)
    xp = jnp.transpose(x, (2, 3, 0, 1)).reshape(hw, N, C)

    bn = 16 if N % 16 == 0 else (8 if N % 8 == 0 else N)
    out = pl.pallas_call(
        functools.partial(_hwnc_body, inv_hw=inv_hw),
        out_shape=jax.ShapeDtypeStruct((N, C), jnp.float32),
        grid=(N // bn,),
        in_specs=[pl.BlockSpec((hw, bn, C), lambda i: (0, i, 0))],
        out_specs=pl.BlockSpec((bn, C), lambda i: (i, 0)),
        compiler_params=pltpu.CompilerParams(
            dimension_semantics=("parallel",),
            vmem_limit_bytes=64 << 20,
        ),
        cost_estimate=pl.CostEstimate(
            flops=N * C * hw, transcendentals=0,
            bytes_accessed=N * C * hw * 4 + N * C * 4),
    )(xp)
    return out.reshape(N, C, 1, 1)


def _nhwc_body(x_ref, o_ref, *, inv_hw):
    o_ref[...] = jnp.sum(x_ref[...], axis=1, keepdims=True,
                         dtype=jnp.float32) * inv_hw


def _gap_fallback(x, N, C, hw):
    inv_hw = 1.0 / float(hw)
    xt = jnp.transpose(x.reshape(N, C, hw), (0, 2, 1))
    out = pl.pallas_call(
        functools.partial(_nhwc_body, inv_hw=inv_hw),
        out_shape=jax.ShapeDtypeStruct((N, 1, C), jnp.float32),
        grid=(N,),
        in_specs=[pl.BlockSpec((1, hw, C), lambda i: (i, 0, 0))],
        out_specs=pl.BlockSpec((1, 1, C), lambda i: (i, 0, 0)),
        compiler_params=pltpu.CompilerParams(
            dimension_semantics=("parallel",),
            vmem_limit_bytes=64 << 20,
        ),
    )(xt)
    return out.reshape(N, C, 1, 1)


def kernel(x):
    N, C, H, W = x.shape
    hw = H * W
    if C % 128 == 0 and N % 8 == 0:
        return _gap_hwnc(x, N, C, hw)
    return _gap_fallback(x, N, C, hw)
